# Initial kernel scaffold; baseline (speedup 1.0000x reference)
import jax
import jax.numpy as jnp
from jax.experimental import pallas as pl


def kernel(x, router_W, router_b, eW1, eb1, eW2, eb2, sW1, sb1, sW2, sb2):
    raise NotImplementedError("write your pallas kernel here")


# speedup vs baseline: 2.2372x; 2.2372x over previous
import functools

import jax
import jax.numpy as jnp
from jax.experimental import pallas as pl
from jax.experimental.pallas import tpu as pltpu


def _moe_body(xf_ref, xb_ref, rW_ref, rb_ref, W1_ref, b1_ref, W2_ref, b2_ref,
              out_ref, loss_ref, comb_ref, cnt_ref, *, T, E, NE, NT, N, K):
    t = pl.program_id(0)
    e = pl.program_id(1)

    @pl.when(e == 0)
    def _route():
        logits = jnp.dot(xf_ref[...], rW_ref[...],
                         preferred_element_type=jnp.float32) + rb_ref[...]
        lane = jax.lax.broadcasted_iota(jnp.int32, (T, E), 1)
        m0 = jnp.max(logits, axis=1, keepdims=True)
        i0 = jnp.argmax(logits, axis=1).reshape(T, 1)
        masked = jnp.where(lane == i0, -jnp.inf, logits)
        m1 = jnp.max(masked, axis=1, keepdims=True)
        i1 = jnp.argmax(masked, axis=1).reshape(T, 1)
        d = jnp.exp(m1 - m0)
        g0 = 1.0 / (1.0 + d)
        g1 = d / (1.0 + d)
        comb_ref[...] = (g0 * (lane == i0).astype(jnp.float32)
                         + g1 * (lane == i1).astype(jnp.float32))
        c0 = jnp.sum((i0 == 0).astype(jnp.float32))
        c1 = jnp.sum((i1 == 1).astype(jnp.float32))

        @pl.when(t == 0)
        def _():
            cnt_ref[0] = c0
            cnt_ref[1] = c1

        @pl.when(t > 0)
        def _():
            cnt_ref[0] += c0
            cnt_ref[1] += c1

    h = jnp.dot(xb_ref[...], W1_ref[0], preferred_element_type=jnp.float32)
    h = h + b1_ref[0]
    h = 0.5 * h * (1.0 + jax.lax.erf(h * 0.7071067811865476))
    y = jnp.dot(h.astype(jnp.bfloat16), W2_ref[0],
                preferred_element_type=jnp.float32)
    y = y + b2_ref[0]

    lane = jax.lax.broadcasted_iota(jnp.int32, (T, E), 1)
    cw_routed = jnp.sum(comb_ref[...] * (lane == e).astype(jnp.float32),
                        axis=1, keepdims=True)
    cw = jnp.where(e < E, cw_routed, 1.0)
    contrib = cw * y

    @pl.when(e == 0)
    def _():
        out_ref[...] = contrib

    @pl.when(e > 0)
    def _():
        out_ref[...] += contrib

    @pl.when(jnp.logical_and(t == NT - 1, e == NE - 1))
    def _loss():
        lane8 = jax.lax.broadcasted_iota(jnp.int32, (1, E), 1)
        ec = jnp.where(lane8 == 0, cnt_ref[0],
                       jnp.where(lane8 == 1, cnt_ref[1], 0.0))
        ec = ec / (N * K) + 1e-08
        loss_ref[...] = (-jnp.sum(ec * jnp.log(ec))).reshape(1, 1)


def kernel(x, router_W, router_b, eW1, eb1, eW2, eb2, sW1, sb1, sW2, sb2):
    B, S, DIM = x.shape
    E, _, F = eW1.shape
    NS = sW1.shape[0]
    K = 2
    N = B * S
    NE = E + NS
    T = 512
    NT = N // T

    xf = x.reshape(N, DIM)
    xb = xf.astype(jnp.bfloat16)
    W1c = jnp.concatenate([eW1, sW1], axis=0).astype(jnp.bfloat16)
    W2c = jnp.concatenate([eW2, sW2], axis=0).astype(jnp.bfloat16)
    b1c = jnp.concatenate([eb1, sb1], axis=0).reshape(NE, 1, F)
    b2c = jnp.concatenate([eb2, sb2], axis=0).reshape(NE, 1, DIM)
    rb = router_b.reshape(1, E)

    body = functools.partial(_moe_body, T=T, E=E, NE=NE, NT=NT, N=N, K=K)
    out, loss = pl.pallas_call(
        body,
        grid=(NT, NE),
        in_specs=[
            pl.BlockSpec((T, DIM), lambda t, e: (t, 0)),
            pl.BlockSpec((T, DIM), lambda t, e: (t, 0)),
            pl.BlockSpec((DIM, E), lambda t, e: (0, 0)),
            pl.BlockSpec((1, E), lambda t, e: (0, 0)),
            pl.BlockSpec((1, DIM, F), lambda t, e: (e, 0, 0)),
            pl.BlockSpec((1, 1, F), lambda t, e: (e, 0, 0)),
            pl.BlockSpec((1, F, DIM), lambda t, e: (e, 0, 0)),
            pl.BlockSpec((1, 1, DIM), lambda t, e: (e, 0, 0)),
        ],
        out_specs=[
            pl.BlockSpec((T, DIM), lambda t, e: (t, 0)),
            pl.BlockSpec((1, 1), lambda t, e: (0, 0)),
        ],
        out_shape=[
            jax.ShapeDtypeStruct((N, DIM), jnp.float32),
            jax.ShapeDtypeStruct((1, 1), jnp.float32),
        ],
        scratch_shapes=[
            pltpu.VMEM((T, E), jnp.float32),
            pltpu.SMEM((2,), jnp.float32),
        ],
    )(xf, xb, router_W, rb, W1c, b1c, W2c, b2c)
    return out.reshape(B, S, DIM), loss[0, 0]


# speedup vs baseline: 3.1498x; 1.4079x over previous
import functools

import jax
import jax.numpy as jnp
from jax.experimental import pallas as pl
from jax.experimental.pallas import tpu as pltpu


def _moe_body(xf_ref, rW_ref, rb_ref, W1_ref, b1_ref, W2_ref, b2_ref,
              out_ref, loss_ref, cnt_ref, *, T, E, NE, NT, N, K):
    t = pl.program_id(0)

    xf = xf_ref[...]
    logits = jnp.dot(xf, rW_ref[...],
                     preferred_element_type=jnp.float32) + rb_ref[...]
    lane = jax.lax.broadcasted_iota(jnp.int32, (T, E), 1)
    m0 = jnp.max(logits, axis=1, keepdims=True)
    i0 = jnp.argmax(logits, axis=1).reshape(T, 1)
    masked = jnp.where(lane == i0, -jnp.inf, logits)
    m1 = jnp.max(masked, axis=1, keepdims=True)
    i1 = jnp.argmax(masked, axis=1).reshape(T, 1)
    d = jnp.exp(m1 - m0)
    g0 = 1.0 / (1.0 + d)
    g1 = d / (1.0 + d)
    comb = (g0 * (lane == i0).astype(jnp.float32)
            + g1 * (lane == i1).astype(jnp.float32))

    c0 = jnp.sum((i0 == 0).astype(jnp.float32))
    c1 = jnp.sum((i1 == 1).astype(jnp.float32))

    @pl.when(t == 0)
    def _():
        cnt_ref[0] = c0
        cnt_ref[1] = c1

    @pl.when(t > 0)
    def _():
        cnt_ref[0] += c0
        cnt_ref[1] += c1

    xb = xf.astype(jnp.bfloat16)
    acc = None
    for e in range(NE):
        h = jnp.dot(xb, W1_ref[e], preferred_element_type=jnp.float32)
        h = h + b1_ref[e]
        h = 0.5 * h * (1.0 + jax.lax.erf(h * 0.7071067811865476))
        y = jnp.dot(h.astype(jnp.bfloat16), W2_ref[e],
                    preferred_element_type=jnp.float32)
        y = y + b2_ref[e]
        if e < E:
            cw = comb[:, e:e + 1]
        else:
            cw = 1.0
        contrib = cw * y
        acc = contrib if acc is None else acc + contrib
    out_ref[...] = acc

    @pl.when(t == NT - 1)
    def _loss():
        lane8 = jax.lax.broadcasted_iota(jnp.int32, (1, E), 1)
        ec = jnp.where(lane8 == 0, cnt_ref[0],
                       jnp.where(lane8 == 1, cnt_ref[1], 0.0))
        ec = ec / (N * K) + 1e-08
        loss_ref[...] = (-jnp.sum(ec * jnp.log(ec))).reshape(1, 1)


def kernel(x, router_W, router_b, eW1, eb1, eW2, eb2, sW1, sb1, sW2, sb2):
    B, S, DIM = x.shape
    E, _, F = eW1.shape
    NS = sW1.shape[0]
    K = 2
    N = B * S
    NE = E + NS
    T = 512
    NT = N // T

    xf = x.reshape(N, DIM)
    W1c = jnp.concatenate([eW1, sW1], axis=0).astype(jnp.bfloat16)
    W2c = jnp.concatenate([eW2, sW2], axis=0).astype(jnp.bfloat16)
    b1c = jnp.concatenate([eb1, sb1], axis=0).reshape(NE, 1, F)
    b2c = jnp.concatenate([eb2, sb2], axis=0).reshape(NE, 1, DIM)
    rb = router_b.reshape(1, E)

    body = functools.partial(_moe_body, T=T, E=E, NE=NE, NT=NT, N=N, K=K)
    out, loss = pl.pallas_call(
        body,
        grid=(NT,),
        in_specs=[
            pl.BlockSpec((T, DIM), lambda t: (t, 0)),
            pl.BlockSpec((DIM, E), lambda t: (0, 0)),
            pl.BlockSpec((1, E), lambda t: (0, 0)),
            pl.BlockSpec((NE, DIM, F), lambda t: (0, 0, 0)),
            pl.BlockSpec((NE, 1, F), lambda t: (0, 0, 0)),
            pl.BlockSpec((NE, F, DIM), lambda t: (0, 0, 0)),
            pl.BlockSpec((NE, 1, DIM), lambda t: (0, 0, 0)),
        ],
        out_specs=[
            pl.BlockSpec((T, DIM), lambda t: (t, 0)),
            pl.BlockSpec((1, 1), lambda t: (0, 0)),
        ],
        out_shape=[
            jax.ShapeDtypeStruct((N, DIM), jnp.float32),
            jax.ShapeDtypeStruct((1, 1), jnp.float32),
        ],
        scratch_shapes=[
            pltpu.SMEM((2,), jnp.float32),
        ],
    )(xf, router_W, rb, W1c, b1c, W2c, b2c)
    return out.reshape(B, S, DIM), loss[0, 0]
